# baseline (speedup 1.0000x reference)
import functools
import math

import jax
import jax.numpy as jnp
from jax import lax
from jax.experimental import pallas as pl
from jax.experimental.pallas import tpu as pltpu
from jax.experimental.pallas import tpu_sc as plsc

B = 16384
D = 32
NC = 2
NS = 16
NW = NC * NS
BPW = B // NW

_mesh = plsc.VectorSubcoreMesh(core_axis_name="c", subcore_axis_name="s")


@functools.partial(
    pl.kernel,
    mesh=_mesh,
    out_type=jax.ShapeDtypeStruct((B,), jnp.float32),
    scratch_types=[
        pltpu.VMEM((BPW,), jnp.int32),
        pltpu.VMEM((2 * BPW,), jnp.int32),
        pltpu.VMEM((BPW, D), jnp.float32),
        pltpu.VMEM((2 * BPW, D), jnp.float32),
        pltpu.VMEM((16 * BPW,), jnp.float32),
        pltpu.VMEM((BPW,), jnp.float32),
        pltpu.SemaphoreType.DMA,
        pltpu.SemaphoreType.DMA,
    ],
    compiler_params=pltpu.CompilerParams(
        needs_layout_passes=False, use_tc_tiling_on_sc=False
    ),
)
def _sc_dots(users_hbm, item_idx_hbm, ut_hbm, it_hbm, out_hbm,
             ui_v, ii_v, u_v, pn_v, t_v, d_v, s1, s2):
    wid = lax.axis_index("s") * NC + lax.axis_index("c")
    base = wid * BPW
    pltpu.sync_copy(users_hbm.at[pl.ds(base, BPW)], ui_v)
    pltpu.sync_copy(item_idx_hbm.at[pl.ds(2 * base, 2 * BPW)], ii_v)
    cu = pltpu.async_copy(ut_hbm.at[ui_v], u_v, s1)
    ci = pltpu.async_copy(it_hbm.at[ii_v], pn_v, s2)
    cu.wait()
    ci.wait()

    lane = lax.iota(jnp.int32, 16)

    lane_off = lane * BPW

    def row_body(b, carry):
        for j in range(4):
            i = b * 4 + j
            u0 = u_v[i, pl.ds(0, 16)]
            u1 = u_v[i, pl.ds(16, 16)]
            p0 = pn_v[i, pl.ds(0, 16)]
            p1 = pn_v[i, pl.ds(16, 16)]
            n0 = pn_v[i + BPW, pl.ds(0, 16)]
            n1 = pn_v[i + BPW, pl.ds(16, 16)]
            s = u0 * (p0 - n0) + u1 * (p1 - n1)
            plsc.store_scatter(t_v, [lane_off + i], s)
        return carry

    lax.fori_loop(0, BPW // 4, row_body, 0)

    def col_body(c, carry):
        acc = t_v[pl.ds(c * 16, 16)]
        for k in range(1, 16):
            acc = acc + t_v[pl.ds(k * BPW + c * 16, 16)]
        d_v[pl.ds(c * 16, 16)] = acc
        return carry

    lax.fori_loop(0, BPW // 16, col_body, 0)
    pltpu.sync_copy(d_v, out_hbm.at[pl.ds(base, BPW)])


_INV_LN2 = 1.0 / math.log(2.0)


def _loss_body(x_ref, o_ref):
    x = x_ref[...]
    t = -x
    sp = jnp.maximum(t, 0.0) + jnp.log1p(jnp.exp(-jnp.abs(t)))
    o_ref[0, 0] = jnp.sum(sp) * _INV_LN2


_loss_call = pl.pallas_call(
    _loss_body,
    out_shape=jax.ShapeDtypeStruct((1, 1), jnp.float32),
    out_specs=pl.BlockSpec(memory_space=pltpu.SMEM),
)


@jax.jit
def kernel(users, pos_items, neg_items, user_table, item_table):
    users = users.astype(jnp.int32)
    pos_items = pos_items.astype(jnp.int32)
    neg_items = neg_items.astype(jnp.int32)
    item_idx = jnp.concatenate(
        [pos_items.reshape(NW, BPW), neg_items.reshape(NW, BPW)], axis=1
    ).reshape(2 * B)
    d = _sc_dots(users, item_idx, user_table, item_table)
    loss = _loss_call(d.reshape(128, 128))
    return loss[0, 0]
